# baseline (speedup 1.0000x reference)
import functools

import jax
import jax.numpy as jnp
from jax import lax
from jax.experimental import pallas as pl
from jax.experimental.pallas import tpu as pltpu
from jax.experimental.pallas import tpu_sc as plsc

_NC, _NS, _LANES = 2, 16, 16
_NW = _NC * _NS
_C = 400
_H = 64
_JU = 8


def _node_table(x_proj, W1, b1):
    N, D = x_proj.shape
    H = W1.shape[0]
    wstk = jnp.stack([W1[:, :D].T, W1[:, D:].T])
    bias = jnp.stack([b1, jnp.zeros((H,), jnp.float32)])[:, None, :]

    we = wstk[:, :, 0::2]
    wo = wstk[:, :, 1::2]
    be = bias[:, :, 0::2]
    bo = bias[:, :, 1::2]

    def mm_kernel(x_ref, we_ref, wo_ref, be_ref, bo_ref, o_ref):
        x = x_ref[...]
        for g in range(2):
            av = (jnp.dot(x, we_ref[g], preferred_element_type=jnp.float32)
                  + be_ref[g]).astype(jnp.bfloat16)
            bv = (jnp.dot(x, wo_ref[g], preferred_element_type=jnp.float32)
                  + bo_ref[g]).astype(jnp.bfloat16)
            alo = lax.convert_element_type(
                lax.bitcast_convert_type(av, jnp.uint16), jnp.uint32)
            bhi = lax.convert_element_type(
                lax.bitcast_convert_type(bv, jnp.uint16), jnp.uint32)
            o_ref[g] = lax.bitcast_convert_type(
                alo | (bhi << jnp.uint32(16)), jnp.int32)

    out = pl.pallas_call(
        mm_kernel,
        out_shape=jax.ShapeDtypeStruct((2, N, H // 2), jnp.int32),
    )(x_proj, we, wo, be, bo)
    return out.reshape(2 * N, H // 2)


def _edge_scores_sc(tab, edge_index, wd, b2v, nwin):
    wpw = nwin // _NW
    hw = _H // 2
    n_nodes = tab.shape[0] // 2
    mesh = plsc.VectorSubcoreMesh(core_axis_name="c", subcore_axis_name="s")

    @functools.partial(
        pl.kernel,
        mesh=mesh,
        compiler_params=pltpu.CompilerParams(needs_layout_passes=False,
                                             use_tc_tiling_on_sc=False),
        out_type=jax.ShapeDtypeStruct((nwin, _C), jnp.float32),
        scratch_types=[
            pltpu.VMEM((2, 2 * _C), jnp.int32),
            pltpu.VMEM((2, 2 * _C, _H // 2), jnp.int32),
            pltpu.VMEM((2, _C), jnp.float32),
            pltpu.VMEM((_H // 2, _LANES), jnp.int32),
            pltpu.VMEM((_LANES,), jnp.float32),
            pltpu.VMEM_SHARED((n_nodes, _H // 2), jnp.int32),
            pltpu.VMEM_SHARED((n_nodes, _H // 2), jnp.int32),
            pltpu.SemaphoreType.DMA,
            pltpu.SemaphoreType.DMA,
            pltpu.SemaphoreType.DMA,
            pltpu.SemaphoreType.DMA,
        ],
    )
    def k(t_hbm, ei_hbm, wd_hbm, b2_hbm, out_hbm,
          idx_v, tg, out_v, wd_v, b2_v, stab_p, stab_q,
          isem, gsem, osem0, osem1):
        wid = lax.axis_index("s") * _NC + lax.axis_index("c")
        pltpu.sync_copy(wd_hbm, wd_v)
        pltpu.sync_copy(b2_hbm, b2_v)
        sid = lax.axis_index("s")
        rows_per = n_nodes // _NS
        r0 = sid * rows_per
        pltpu.sync_copy(t_hbm.at[pl.ds(r0, rows_per)],
                        stab_p.at[pl.ds(r0, rows_per)])
        pltpu.sync_copy(t_hbm.at[pl.ds(n_nodes + r0, rows_per)],
                        stab_q.at[pl.ds(r0, rows_per)])
        plsc.subcore_barrier()
        w0 = wid * wpw

        class _Pair:
            def __init__(self, h1, h2):
                self.h1, self.h2 = h1, h2

            def start(self):
                self.h1.start()
                self.h2.start()

            def wait(self):
                self.h1.wait()
                self.h2.wait()

        def fire_idx(win, p):
            e0 = win * _C
            return _Pair(
                pltpu.make_async_copy(ei_hbm.at[0, pl.ds(e0, _C)],
                                      idx_v.at[p, pl.ds(0, _C)], isem),
                pltpu.make_async_copy(ei_hbm.at[1, pl.ds(e0, _C)],
                                      idx_v.at[p, pl.ds(_C, _C)], isem))

        def fire_gather(p):
            return _Pair(
                pltpu.make_async_copy(stab_p.at[idx_v.at[p, pl.ds(0, _C)]],
                                      tg.at[p, pl.ds(0, _C)], gsem),
                pltpu.make_async_copy(stab_q.at[idx_v.at[p, pl.ds(_C, _C)]],
                                      tg.at[p, pl.ds(_C, _C)], gsem))

        def fire_out(win, p, osem):
            return pltpu.make_async_copy(out_v.at[p], out_hbm.at[win], osem)

        def compute(p):
            dvec = lax.iota(jnp.int32, _LANES)
            tgp = tg.at[p]

            def pair(e0):
                evec0 = e0 + dvec
                evec0q = evec0 + _C
                evec1 = evec0 + _LANES
                evec1q = evec1 + _C

                def jblock(jb, accs):
                    a0, a1 = accs
                    jbase = jb * _JU
                    for jj in range(_JU):
                        jrow = jbase + jj
                        colvec = (dvec + jrow) & (hw - 1)
                        wv = plsc.bitcast(wd_v[jrow], jnp.bfloat16)
                        pw0 = plsc.load_gather(tgp, [evec0, colvec])
                        qw0 = plsc.load_gather(tgp, [evec0q, colvec])
                        pw1 = plsc.load_gather(tgp, [evec1, colvec])
                        qw1 = plsc.load_gather(tgp, [evec1q, colvec])
                        a0 = a0 + jnp.maximum(
                            plsc.bitcast(pw0, jnp.bfloat16)
                            + plsc.bitcast(qw0, jnp.bfloat16),
                            jnp.bfloat16(0)) * wv
                        a1 = a1 + jnp.maximum(
                            plsc.bitcast(pw1, jnp.bfloat16)
                            + plsc.bitcast(qw1, jnp.bfloat16),
                            jnp.bfloat16(0)) * wv
                    return a0, a1

                zinit = jnp.zeros((2 * _LANES,), jnp.bfloat16)
                a0, a1 = lax.fori_loop(0, hw // _JU, jblock, (zinit, zinit))
                for a, ee in ((a0, e0), (a1, e0 + _LANES)):
                    pe, po = plsc.unpack(a, format=plsc.PackFormat.INTERLEAVED)
                    z = pe + po + b2_v[...]
                    out_v.at[p][pl.ds(ee, _LANES)] = 1.0 / (1.0 + jnp.exp(-z))

            @pl.loop(0, _C - _LANES, step=2 * _LANES)
            def _grp(e0):
                pair(e0)

            pair(jnp.int32(_C - 2 * _LANES))

        fire_idx(w0, 0).start()
        fire_idx(w0, 0).wait()
        fire_gather(0).start()
        fire_idx(w0 + 1, 1).start()
        fire_gather(0).wait()

        def body(i, p, osem):
            win = w0 + i

            @pl.when(i + 1 < wpw)
            def _():
                fire_idx(win + 1, 1 - p).wait()
                fire_gather(1 - p).start()

            @pl.when(i + 2 < wpw)
            def _():
                fire_idx(win + 2, p).start()

            @pl.when(i >= 2)
            def _():
                fire_out(win - 2, p, osem).wait()

            compute(p)
            fire_out(win, p, osem).start()

            @pl.when(i + 1 < wpw)
            def _():
                fire_gather(1 - p).wait()

        @pl.loop(0, wpw - 1, step=2)
        def _pair(i):
            body(i, 0, osem0)
            body(i + 1, 1, osem1)

        body(jnp.int32(wpw - 1), (wpw - 1) % 2, osem0 if wpw % 2 else osem1)
        fire_out(w0 + wpw - 2, wpw % 2, osem1 if wpw % 2 else osem0).wait()
        fire_out(w0 + wpw - 1, (wpw - 1) % 2, osem0 if wpw % 2 else osem1).wait()

    return k(tab, edge_index, wd, b2v)


def kernel(x_proj, edge_index, chunk_size, W1, b1, W2, b2):
    del chunk_size
    N = x_proj.shape[0]
    E = edge_index.shape[1]
    assert E % (_NW * _C) == 0
    nwin = E // _C

    tab = _node_table(x_proj, W1, b1)

    hw = _H // 2
    c = (jnp.arange(hw)[:, None] + jnp.arange(_LANES)[None, :]) % hw
    wpair = jnp.stack([W2[0][2 * c], W2[0][2 * c + 1]], axis=-1)
    wd = lax.bitcast_convert_type(wpair.astype(jnp.bfloat16), jnp.int32)
    b2v = jnp.broadcast_to(b2, (_LANES,)).astype(jnp.float32)

    out = _edge_scores_sc(tab, edge_index, wd, b2v, nwin)
    return out.reshape(E)
